# baseline (speedup 1.0000x reference)
import functools

import jax
import jax.numpy as jnp
from jax import lax
from jax.experimental import pallas as pl
from jax.experimental.pallas import tpu as pltpu
from jax.experimental.pallas import tpu_sc as plsc

N_CH = 144
Z = 1024
DIM_OUT = 10440
NNZ = 82944

NC = 2
NS = 16
NW = NC * NS
ZT = 8
NZC = Z // ZT
ROUNDS = NZC // NW
BATCH = 128
SB = 4
NSB = NNZ // (SB * BATCH)
ACC_W = ZT * DIM_OUT


def _sc_spmm(ftz, meta, vals_a):
    mesh = plsc.VectorSubcoreMesh(core_axis_name="c", subcore_axis_name="s")

    @functools.partial(
        pl.kernel,
        out_type=jax.ShapeDtypeStruct((NZC, ACC_W), jnp.float32),
        mesh=mesh,
        scratch_types=[
            pltpu.VMEM((N_CH * ZT,), jnp.float32),
            pltpu.VMEM((2, SB, 2, BATCH), jnp.int32),
            pltpu.VMEM((2, SB, BATCH), jnp.float32),
            pltpu.VMEM((ACC_W,), jnp.float32),
            pltpu.SemaphoreType.DMA,
            pltpu.SemaphoreType.DMA,
        ],
        compiler_params=pltpu.CompilerParams(needs_layout_passes=False),
    )
    def body(ftz_hbm, m_hbm, v_hbm, out_hbm, ft_t, m_v, v_v, acc_t,
             sem0, sem1):
        cidx = lax.axis_index("c")
        sidx = lax.axis_index("s")
        wid = sidx * NC + cidx
        sems = (sem0, sem1)
        zero16 = jnp.zeros((16,), jnp.float32)

        def process_batch(buf, k):
            for g0 in range(0, BATCH // 16, 2):
                pvs, rvs, vvs = [], [], []
                for g in (g0, g0 + 1):
                    pvs.append(m_v[buf, k, 0, pl.ds(g * 16, 16)])
                    rvs.append(m_v[buf, k, 1, pl.ds(g * 16, 16)])
                    vvs.append(v_v[buf, k, pl.ds(g * 16, 16)])
                ias = [lax.shift_right_logical(pv, 8) for pv in pvs]
                ibs = [lax.bitwise_and(pv, 255) for pv in pvs]
                gas = [[plsc.load_gather(ft_t, [ia + (zi * N_CH)])
                        for zi in range(ZT)] for ia in ias]
                gbs = [[plsc.load_gather(ft_t, [ib + (zi * N_CH)])
                        for zi in range(ZT)] for ib in ibs]
                prods = [[ga * gb * vv for ga, gb in zip(gac, gbc)]
                         for gac, gbc, vv in zip(gas, gbs, vvs)]
                for p in range(2):
                    for zi in range(ZT):
                        plsc.addupdate_scatter(
                            acc_t, [rvs[p] + (zi * DIM_OUT)], prods[p][zi])

        def sb_body(buf):
            def k_body(k, carry):
                process_batch(buf, k)
                return carry
            lax.fori_loop(0, SB, k_body, 0)

        def chunk_body(t, carry):
            zc = t * NW + wid

            def zero_body(q, carry0):
                acc_t[pl.ds(q * 16, 16)] = zero16
                return carry0

            lax.fori_loop(0, ACC_W // 16, zero_body, 0)

            pltpu.sync_copy(ftz_hbm.at[zc], ft_t)

            pltpu.async_copy(m_hbm.at[0], m_v.at[0], sem0)
            pltpu.async_copy(v_hbm.at[0], v_v.at[0], sem0)

            def pair_body(h, carry1):
                sb0 = h * 2
                for par in range(2):
                    sbi = sb0 + par
                    pltpu.make_async_copy(
                        m_hbm.at[sbi], m_v.at[par], sems[par]).wait()
                    pltpu.make_async_copy(
                        v_hbm.at[sbi], v_v.at[par], sems[par]).wait()
                    pltpu.async_copy(
                        m_hbm.at[sbi + 1], m_v.at[1 - par], sems[1 - par])
                    pltpu.async_copy(
                        v_hbm.at[sbi + 1], v_v.at[1 - par], sems[1 - par])
                    sb_body(par)
                return carry1

            lax.fori_loop(0, (NSB - 2) // 2, pair_body, 0)
            pltpu.make_async_copy(m_hbm.at[NSB - 2], m_v.at[0], sem0).wait()
            pltpu.make_async_copy(v_hbm.at[NSB - 2], v_v.at[0], sem0).wait()
            pltpu.async_copy(m_hbm.at[NSB - 1], m_v.at[1], sem1)
            pltpu.async_copy(v_hbm.at[NSB - 1], v_v.at[1], sem1)
            sb_body(0)
            pltpu.make_async_copy(m_hbm.at[NSB - 1], m_v.at[1], sem1).wait()
            pltpu.make_async_copy(v_hbm.at[NSB - 1], v_v.at[1], sem1).wait()
            sb_body(1)

            pltpu.sync_copy(acc_t, out_hbm.at[zc])
            return carry

        lax.fori_loop(0, ROUNDS, chunk_body, 0)

    return body(ftz, meta, vals_a)


def kernel(features, mix_rows, mix_cols, mix_vals):
    f = features.reshape(-1, N_CH)
    ftz = f.reshape(NZC, ZT * N_CH)

    cols = mix_cols.astype(jnp.int32)
    i_idx = cols // N_CH
    j_idx = cols - i_idx * N_CH
    packed = (i_idx << 8) | j_idx
    rows = mix_rows.astype(jnp.int32)
    vals = mix_vals.astype(jnp.float32)

    im = i_idx & 15
    jm = j_idx & 15
    rm = rows & 15
    order = jnp.argsort((im << 8) | (((jm - im) & 15) << 4) | ((rm - im) & 15))
    g_total = NNZ // 16

    def lanefy(x):
        return x[order].reshape(16, g_total).T.reshape(-1)

    meta = jnp.stack([lanefy(packed).reshape(NSB, SB, BATCH),
                      lanefy(rows).reshape(NSB, SB, BATCH)], axis=2)
    vals_a = lanefy(vals).reshape(NSB, SB, BATCH)

    out_d = _sc_spmm(ftz, meta, vals_a)
    return out_d.reshape(Z, DIM_OUT)


# speedup vs baseline: 1.0106x; 1.0106x over previous
import functools

import jax
import jax.numpy as jnp
from jax import lax
from jax.experimental import pallas as pl
from jax.experimental.pallas import tpu as pltpu
from jax.experimental.pallas import tpu_sc as plsc

N_CH = 144
Z = 1024
DIM_OUT = 10440
NNZ = 82944

NC = 2
NS = 16
NW = NC * NS
ZT = 8
NZC = Z // ZT
ROUNDS = NZC // NW
BATCH = 128
SB = 4
NSB = NNZ // (SB * BATCH)
ACC_W = ZT * DIM_OUT
FUSE = 4


def _sc_spmm(ftz, meta, vals_a):
    mesh = plsc.VectorSubcoreMesh(core_axis_name="c", subcore_axis_name="s")

    @functools.partial(
        pl.kernel,
        out_type=jax.ShapeDtypeStruct((NZC, ACC_W), jnp.float32),
        mesh=mesh,
        scratch_types=[
            pltpu.VMEM((N_CH * ZT,), jnp.float32),
            pltpu.VMEM((2, SB, 2, BATCH), jnp.int32),
            pltpu.VMEM((2, SB, BATCH), jnp.float32),
            pltpu.VMEM((ACC_W,), jnp.float32),
            pltpu.SemaphoreType.DMA,
            pltpu.SemaphoreType.DMA,
        ],
        compiler_params=pltpu.CompilerParams(needs_layout_passes=False),
    )
    def body(ftz_hbm, m_hbm, v_hbm, out_hbm, ft_t, m_v, v_v, acc_t,
             sem0, sem1):
        cidx = lax.axis_index("c")
        sidx = lax.axis_index("s")
        wid = sidx * NC + cidx
        sems = (sem0, sem1)
        zero16 = jnp.zeros((16,), jnp.float32)

        def process_batch(buf, k):
            for g0 in range(0, BATCH // 16, FUSE):
                pvs, rvs, vvs = [], [], []
                for g in range(g0, g0 + FUSE):
                    pvs.append(m_v[buf, k, 0, pl.ds(g * 16, 16)])
                    rvs.append(m_v[buf, k, 1, pl.ds(g * 16, 16)])
                    vvs.append(v_v[buf, k, pl.ds(g * 16, 16)])
                ias = [lax.shift_right_logical(pv, 8) for pv in pvs]
                ibs = [lax.bitwise_and(pv, 255) for pv in pvs]
                gas = [[plsc.load_gather(ft_t, [ia + (zi * N_CH)])
                        for zi in range(ZT)] for ia in ias]
                gbs = [[plsc.load_gather(ft_t, [ib + (zi * N_CH)])
                        for zi in range(ZT)] for ib in ibs]
                prods = [[ga * gb * vv for ga, gb in zip(gac, gbc)]
                         for gac, gbc, vv in zip(gas, gbs, vvs)]
                for p in range(FUSE):
                    for zi in range(ZT):
                        plsc.addupdate_scatter(
                            acc_t, [rvs[p] + (zi * DIM_OUT)], prods[p][zi])

        def sb_body(buf):
            def k_body(k, carry):
                process_batch(buf, k)
                return carry
            lax.fori_loop(0, SB, k_body, 0)

        def chunk_body(t, carry):
            zc = t * NW + wid

            def zero_body(q, carry0):
                acc_t[pl.ds(q * 16, 16)] = zero16
                return carry0

            lax.fori_loop(0, ACC_W // 16, zero_body, 0)

            pltpu.sync_copy(ftz_hbm.at[zc], ft_t)

            pltpu.async_copy(m_hbm.at[0], m_v.at[0], sem0)
            pltpu.async_copy(v_hbm.at[0], v_v.at[0], sem0)

            def pair_body(h, carry1):
                sb0 = h * 2
                for par in range(2):
                    sbi = sb0 + par
                    pltpu.make_async_copy(
                        m_hbm.at[sbi], m_v.at[par], sems[par]).wait()
                    pltpu.make_async_copy(
                        v_hbm.at[sbi], v_v.at[par], sems[par]).wait()
                    pltpu.async_copy(
                        m_hbm.at[sbi + 1], m_v.at[1 - par], sems[1 - par])
                    pltpu.async_copy(
                        v_hbm.at[sbi + 1], v_v.at[1 - par], sems[1 - par])
                    sb_body(par)
                return carry1

            lax.fori_loop(0, (NSB - 2) // 2, pair_body, 0)
            pltpu.make_async_copy(m_hbm.at[NSB - 2], m_v.at[0], sem0).wait()
            pltpu.make_async_copy(v_hbm.at[NSB - 2], v_v.at[0], sem0).wait()
            pltpu.async_copy(m_hbm.at[NSB - 1], m_v.at[1], sem1)
            pltpu.async_copy(v_hbm.at[NSB - 1], v_v.at[1], sem1)
            sb_body(0)
            pltpu.make_async_copy(m_hbm.at[NSB - 1], m_v.at[1], sem1).wait()
            pltpu.make_async_copy(v_hbm.at[NSB - 1], v_v.at[1], sem1).wait()
            sb_body(1)

            pltpu.sync_copy(acc_t, out_hbm.at[zc])
            return carry

        lax.fori_loop(0, ROUNDS, chunk_body, 0)

    return body(ftz, meta, vals_a)


def kernel(features, mix_rows, mix_cols, mix_vals):
    f = features.reshape(-1, N_CH)
    ftz = f.reshape(NZC, ZT * N_CH)

    cols = mix_cols.astype(jnp.int32)
    i_idx = cols // N_CH
    j_idx = cols - i_idx * N_CH
    packed = (i_idx << 8) | j_idx
    rows = mix_rows.astype(jnp.int32)
    vals = mix_vals.astype(jnp.float32)

    im = i_idx & 15
    jm = j_idx & 15
    order = jnp.argsort((im << 4) | ((jm - im) & 15))
    g_total = NNZ // 16

    def lanefy(x):
        return x[order].reshape(16, g_total).T.reshape(-1)

    meta = jnp.stack([lanefy(packed).reshape(NSB, SB, BATCH),
                      lanefy(rows).reshape(NSB, SB, BATCH)], axis=2)
    vals_a = lanefy(vals).reshape(NSB, SB, BATCH)

    out_d = _sc_spmm(ftz, meta, vals_a)
    return out_d.reshape(Z, DIM_OUT)


# speedup vs baseline: 1.1529x; 1.1407x over previous
import functools

import jax
import jax.numpy as jnp
from jax import lax
from jax.experimental import pallas as pl
from jax.experimental.pallas import tpu as pltpu
from jax.experimental.pallas import tpu_sc as plsc

N_CH = 144
Z = 1024
DIM_OUT = 10440
NNZ = 82944

NC = 2
NS = 16
NW = NC * NS
ZT = 8
NZC = Z // ZT
ROUNDS = NZC // NW
BATCH = 128
SB = 4
NSB = NNZ // (SB * BATCH)
ACC_W = ZT * DIM_OUT
FUSE = 2


def _sc_spmm(ftz, meta, vals_a):
    mesh = plsc.VectorSubcoreMesh(core_axis_name="c", subcore_axis_name="s")

    @functools.partial(
        pl.kernel,
        out_type=jax.ShapeDtypeStruct((NZC, ACC_W), jnp.float32),
        mesh=mesh,
        scratch_types=[
            pltpu.VMEM((N_CH * ZT,), jnp.float32),
            pltpu.VMEM((2, SB, 2, BATCH), jnp.int32),
            pltpu.VMEM((2, SB, BATCH), jnp.float32),
            pltpu.VMEM((ACC_W,), jnp.float32),
            pltpu.SemaphoreType.DMA,
            pltpu.SemaphoreType.DMA,
        ],
        compiler_params=pltpu.CompilerParams(needs_layout_passes=False),
    )
    def body(ftz_hbm, m_hbm, v_hbm, out_hbm, ft_t, m_v, v_v, acc_t,
             sem0, sem1):
        cidx = lax.axis_index("c")
        sidx = lax.axis_index("s")
        wid = sidx * NC + cidx
        sems = (sem0, sem1)
        zero16 = jnp.zeros((16,), jnp.float32)

        def process_batch(buf, k):
            for g0 in range(0, BATCH // 16, FUSE):
                pvs, rvs, vvs = [], [], []
                for g in range(g0, g0 + FUSE):
                    pvs.append(m_v[buf, k, 0, pl.ds(g * 16, 16)])
                    rvs.append(m_v[buf, k, 1, pl.ds(g * 16, 16)])
                    vvs.append(v_v[buf, k, pl.ds(g * 16, 16)])
                ias = [lax.shift_right_logical(pv, 8) for pv in pvs]
                ibs = [lax.bitwise_and(pv, 255) for pv in pvs]
                gas = [[plsc.load_gather(ft_t, [ia + (zi * N_CH)])
                        for zi in range(ZT)] for ia in ias]
                gbs = [[plsc.load_gather(ft_t, [ib + (zi * N_CH)])
                        for zi in range(ZT)] for ib in ibs]
                prods = [[ga * gb * vv for ga, gb in zip(gac, gbc)]
                         for gac, gbc, vv in zip(gas, gbs, vvs)]
                for p in range(FUSE):
                    for zi in range(ZT):
                        plsc.addupdate_scatter(
                            acc_t, [rvs[p] + (zi * DIM_OUT)], prods[p][zi])

        def sb_body(buf):
            def k_body(k, carry):
                process_batch(buf, k)
                return carry
            lax.fori_loop(0, SB, k_body, 0)

        def chunk_body(t, carry):
            zc = t * NW + wid

            def zero_body(q, carry0):
                acc_t[pl.ds(q * 16, 16)] = zero16
                return carry0

            lax.fori_loop(0, ACC_W // 16, zero_body, 0, unroll=16)

            pltpu.sync_copy(ftz_hbm.at[zc], ft_t)

            pltpu.async_copy(m_hbm.at[0], m_v.at[0], sem0)
            pltpu.async_copy(v_hbm.at[0], v_v.at[0], sem0)

            def pair_body(h, carry1):
                sb0 = h * 2
                for par in range(2):
                    sbi = sb0 + par
                    pltpu.make_async_copy(
                        m_hbm.at[sbi], m_v.at[par], sems[par]).wait()
                    pltpu.make_async_copy(
                        v_hbm.at[sbi], v_v.at[par], sems[par]).wait()
                    pltpu.async_copy(
                        m_hbm.at[sbi + 1], m_v.at[1 - par], sems[1 - par])
                    pltpu.async_copy(
                        v_hbm.at[sbi + 1], v_v.at[1 - par], sems[1 - par])
                    sb_body(par)
                return carry1

            lax.fori_loop(0, (NSB - 2) // 2, pair_body, 0)
            pltpu.make_async_copy(m_hbm.at[NSB - 2], m_v.at[0], sem0).wait()
            pltpu.make_async_copy(v_hbm.at[NSB - 2], v_v.at[0], sem0).wait()
            pltpu.async_copy(m_hbm.at[NSB - 1], m_v.at[1], sem1)
            pltpu.async_copy(v_hbm.at[NSB - 1], v_v.at[1], sem1)
            sb_body(0)
            pltpu.make_async_copy(m_hbm.at[NSB - 1], m_v.at[1], sem1).wait()
            pltpu.make_async_copy(v_hbm.at[NSB - 1], v_v.at[1], sem1).wait()
            sb_body(1)

            pltpu.sync_copy(acc_t, out_hbm.at[zc])
            return carry

        lax.fori_loop(0, ROUNDS, chunk_body, 0)

    return body(ftz, meta, vals_a)


def kernel(features, mix_rows, mix_cols, mix_vals):
    f = features.reshape(-1, N_CH)
    ftz = f.reshape(NZC, ZT * N_CH)

    cols = mix_cols.astype(jnp.int32)
    i_idx = cols // N_CH
    j_idx = cols - i_idx * N_CH
    packed = (i_idx << 8) | j_idx
    rows = mix_rows.astype(jnp.int32)
    vals = mix_vals.astype(jnp.float32)

    im = i_idx & 15
    jm = j_idx & 15
    order = jnp.argsort((im << 4) | ((jm - im) & 15))
    g_total = NNZ // 16

    def lanefy(x):
        return x[order].reshape(16, g_total).T.reshape(-1)

    meta = jnp.stack([lanefy(packed).reshape(NSB, SB, BATCH),
                      lanefy(rows).reshape(NSB, SB, BATCH)], axis=2)
    vals_a = lanefy(vals).reshape(NSB, SB, BATCH)

    out_d = _sc_spmm(ftz, meta, vals_a)
    return out_d.reshape(Z, DIM_OUT)


# speedup vs baseline: 1.1530x; 1.0002x over previous
import functools

import jax
import jax.numpy as jnp
from jax import lax
from jax.experimental import pallas as pl
from jax.experimental.pallas import tpu as pltpu
from jax.experimental.pallas import tpu_sc as plsc

N_CH = 144
Z = 1024
DIM_OUT = 10440
NNZ = 82944

NC = 2
NS = 16
NW = NC * NS
ZT = 8
NZC = Z // ZT
ROUNDS = NZC // NW
BATCH = 128
SB = 4
NSB = NNZ // (SB * BATCH)
R_STRIDE = 10442
ACC_W = ZT * R_STRIDE
FUSE = 2


def _sc_spmm(ftz, meta, vals_a):
    mesh = plsc.VectorSubcoreMesh(core_axis_name="c", subcore_axis_name="s")

    @functools.partial(
        pl.kernel,
        out_type=jax.ShapeDtypeStruct((NZC, ACC_W), jnp.float32),
        mesh=mesh,
        scratch_types=[
            pltpu.VMEM((N_CH * ZT,), jnp.float32),
            pltpu.VMEM((2, SB, 2, BATCH), jnp.int32),
            pltpu.VMEM((2, SB, BATCH), jnp.float32),
            pltpu.VMEM((ACC_W,), jnp.float32),
            pltpu.SemaphoreType.DMA,
            pltpu.SemaphoreType.DMA,
        ],
        compiler_params=pltpu.CompilerParams(needs_layout_passes=False),
    )
    def body(ftz_hbm, m_hbm, v_hbm, out_hbm, ft_t, m_v, v_v, acc_t,
             sem0, sem1):
        cidx = lax.axis_index("c")
        sidx = lax.axis_index("s")
        wid = sidx * NC + cidx
        sems = (sem0, sem1)
        zero16 = jnp.zeros((16,), jnp.float32)

        def process_batch(buf, k):
            for g0 in range(0, BATCH // 16, FUSE):
                pvs, rvs, vvs = [], [], []
                for g in range(g0, g0 + FUSE):
                    pvs.append(m_v[buf, k, 0, pl.ds(g * 16, 16)])
                    rvs.append(m_v[buf, k, 1, pl.ds(g * 16, 16)])
                    vvs.append(v_v[buf, k, pl.ds(g * 16, 16)])
                ias = [lax.shift_right_logical(pv, 8) for pv in pvs]
                ibs = [lax.bitwise_and(pv, 255) for pv in pvs]
                gas = [[plsc.load_gather(ft_t, [ia + (zi * N_CH)])
                        for zi in range(ZT)] for ia in ias]
                gbs = [[plsc.load_gather(ft_t, [ib + (zi * N_CH)])
                        for zi in range(ZT)] for ib in ibs]
                prods = [[ga * gb * vv for ga, gb in zip(gac, gbc)]
                         for gac, gbc, vv in zip(gas, gbs, vvs)]
                for p in range(FUSE):
                    for zi in range(ZT):
                        plsc.addupdate_scatter(
                            acc_t, [rvs[p] + (zi * R_STRIDE)], prods[p][zi])

        def sb_body(buf):
            def k_body(k, carry):
                process_batch(buf, k)
                return carry
            lax.fori_loop(0, SB, k_body, 0)

        def chunk_body(t, carry):
            zc = t * NW + wid

            def zero_body(q, carry0):
                acc_t[pl.ds(q * 16, 16)] = zero16
                return carry0

            lax.fori_loop(0, ACC_W // 16, zero_body, 0, unroll=16)

            pltpu.sync_copy(ftz_hbm.at[zc], ft_t)

            pltpu.async_copy(m_hbm.at[0], m_v.at[0], sem0)
            pltpu.async_copy(v_hbm.at[0], v_v.at[0], sem0)

            def pair_body(h, carry1):
                sb0 = h * 2
                for par in range(2):
                    sbi = sb0 + par
                    pltpu.make_async_copy(
                        m_hbm.at[sbi], m_v.at[par], sems[par]).wait()
                    pltpu.make_async_copy(
                        v_hbm.at[sbi], v_v.at[par], sems[par]).wait()
                    pltpu.async_copy(
                        m_hbm.at[sbi + 1], m_v.at[1 - par], sems[1 - par])
                    pltpu.async_copy(
                        v_hbm.at[sbi + 1], v_v.at[1 - par], sems[1 - par])
                    sb_body(par)
                return carry1

            lax.fori_loop(0, (NSB - 2) // 2, pair_body, 0)
            pltpu.make_async_copy(m_hbm.at[NSB - 2], m_v.at[0], sem0).wait()
            pltpu.make_async_copy(v_hbm.at[NSB - 2], v_v.at[0], sem0).wait()
            pltpu.async_copy(m_hbm.at[NSB - 1], m_v.at[1], sem1)
            pltpu.async_copy(v_hbm.at[NSB - 1], v_v.at[1], sem1)
            sb_body(0)
            pltpu.make_async_copy(m_hbm.at[NSB - 1], m_v.at[1], sem1).wait()
            pltpu.make_async_copy(v_hbm.at[NSB - 1], v_v.at[1], sem1).wait()
            sb_body(1)

            pltpu.sync_copy(acc_t, out_hbm.at[zc])
            return carry

        lax.fori_loop(0, ROUNDS, chunk_body, 0)

    return body(ftz, meta, vals_a)


def kernel(features, mix_rows, mix_cols, mix_vals):
    f = features.reshape(-1, N_CH)
    ftz = f.reshape(NZC, ZT * N_CH)

    cols = mix_cols.astype(jnp.int32)
    i_idx = cols // N_CH
    j_idx = cols - i_idx * N_CH
    packed = (i_idx << 8) | j_idx
    rows = mix_rows.astype(jnp.int32)
    vals = mix_vals.astype(jnp.float32)

    im = i_idx & 15
    jm = j_idx & 15
    order = jnp.argsort((im << 4) | ((jm - im) & 15))
    g_total = NNZ // 16

    def lanefy(x):
        return x[order].reshape(16, g_total).T.reshape(-1)

    meta = jnp.stack([lanefy(packed).reshape(NSB, SB, BATCH),
                      lanefy(rows).reshape(NSB, SB, BATCH)], axis=2)
    vals_a = lanefy(vals).reshape(NSB, SB, BATCH)

    out_d = _sc_spmm(ftz, meta, vals_a)
    return out_d.reshape(Z, R_STRIDE)[:, :DIM_OUT]
